# Initial kernel scaffold; baseline (speedup 1.0000x reference)
import functools

import jax
import jax.numpy as jnp
from jax import lax
from jax.experimental import pallas as pl
from jax.experimental.pallas import tpu as pltpu
from jax.experimental.pallas import tpu_sc as plsc

N = 50000
M = 25000
C = 128
E = 400000
K = 27
G = 8
KN = K * N

BN = 400
BM = 200
EW = 128
NWIN = E // EW
MP = 25600
MSUB = MP // 16
DW = 40
NDWIN = M // DW
_EPS = 1e-5


def _gn_tc(h, gamma, beta, groups):
    n, c = h.shape
    cs = c // groups
    gi = lax.broadcasted_iota(jnp.int32, (c, groups), 0) // cs
    gj = lax.broadcasted_iota(jnp.int32, (c, groups), 1)
    Gm = (gi == gj).astype(jnp.float32)
    ti = lax.broadcasted_iota(jnp.int32, (groups, c), 0)
    tj = lax.broadcasted_iota(jnp.int32, (groups, c), 1) // cs
    GmT = (ti == tj).astype(jnp.float32)
    inv_cs = 1.0 / cs
    mu = jnp.dot(h, Gm, preferred_element_type=jnp.float32) * inv_cs
    ex2 = jnp.dot(h * h, Gm, preferred_element_type=jnp.float32) * inv_cs
    var = ex2 - mu * mu
    rstd = lax.rsqrt(var + _EPS)
    mub = jnp.dot(mu, GmT, preferred_element_type=jnp.float32)
    rstdb = jnp.dot(rstd, GmT, preferred_element_type=jnp.float32)
    return (h - mub) * rstdb * gamma + beta


def _stage_a_body(x_ref, w1_ref, g1_ref, b1_ref, w2_ref, xw_ref):
    h = jnp.dot(x_ref[...], w1_ref[...], preferred_element_type=jnp.float32)
    h = _gn_tc(h, g1_ref[...], b1_ref[...], G)
    h = jnp.where(h >= 0, h, 0.01 * h)
    for k in range(K):
        y = jnp.dot(h, w2_ref[k], preferred_element_type=jnp.float32)
        xw_ref[0, k] = y[:, :64]
        xw_ref[1, k] = y[:, 64:]


def _stage_a(x, W1, g1, b1, W2):
    return pl.pallas_call(
        _stage_a_body,
        grid=(N // BN,),
        in_specs=[
            pl.BlockSpec((BN, C), lambda i: (i, 0)),
            pl.BlockSpec((C, C), lambda i: (0, 0)),
            pl.BlockSpec((1, C), lambda i: (0, 0)),
            pl.BlockSpec((1, C), lambda i: (0, 0)),
            pl.BlockSpec((K, C, C), lambda i: (0, 0, 0)),
        ],
        out_specs=pl.BlockSpec((2, K, BN, 64), lambda i: (0, 0, i, 0)),
        out_shape=jax.ShapeDtypeStruct((2, K, N, 64), jnp.float32),
    )(x, W1, g1.reshape(1, C), b1.reshape(1, C), W2)


def _edge_kernel(in_idx, koff, out_idx, xw2, zeros_init):
    mesh = plsc.VectorSubcoreMesh(core_axis_name="c", subcore_axis_name="s")

    @functools.partial(
        pl.kernel,
        out_type=jax.ShapeDtypeStruct((2, MP, 64), jnp.float32),
        mesh=mesh,
        scratch_types=[
            pltpu.VMEM((EW,), jnp.int32),
            pltpu.VMEM((EW,), jnp.int32),
            pltpu.VMEM((EW,), jnp.int32),
            pltpu.VMEM((EW,), jnp.int32),
            pltpu.VMEM((EW, 64), jnp.float32),
            pltpu.VMEM_SHARED((MP, 64), jnp.float32),
        ],
    )
    def body(ii_hbm, kf_hbm, oi_hbm, xw_hbm, z_hbm, out_hbm,
             ii_v, kf_v, oi_v, gi_v, rows_v, acc):
        c = lax.axis_index("c")
        s = lax.axis_index("s")
        pltpu.sync_copy(z_hbm, acc.at[pl.ds(s * MSUB, MSUB)])
        plsc.subcore_barrier()
        half_base = c * jnp.int32(KN)

        @pl.loop(0, 196)
        def _win(t):
            w = s + t * 16

            @pl.when(w < NWIN)
            def _():
                base = w * EW
                pltpu.sync_copy(ii_hbm.at[pl.ds(base, EW)], ii_v)
                pltpu.sync_copy(kf_hbm.at[pl.ds(base, EW)], kf_v)
                pltpu.sync_copy(oi_hbm.at[pl.ds(base, EW)], oi_v)
                for i in range(EW // 16):
                    sl = pl.ds(i * 16, 16)
                    gi_v[sl] = kf_v[sl] * jnp.int32(N) + ii_v[sl] + half_base
                pltpu.sync_copy(xw_hbm.at[gi_v], rows_v)
                pltpu.sync_copy(rows_v, acc.at[oi_v], add=True)

        plsc.subcore_barrier()
        pltpu.sync_copy(acc.at[pl.ds(s * MSUB, MSUB)],
                        out_hbm.at[c].at[pl.ds(s * MSUB, MSUB)])

    return body(in_idx, koff, out_idx, xw2, zeros_init)


def _ds_kernel(ds_idx, x):
    mesh = plsc.VectorSubcoreMesh(core_axis_name="c", subcore_axis_name="s")

    @functools.partial(
        pl.kernel,
        out_type=jax.ShapeDtypeStruct((M, C), jnp.float32),
        mesh=mesh,
        scratch_types=[
            pltpu.VMEM((DW,), jnp.int32),
            pltpu.VMEM((DW, C), jnp.float32),
        ],
    )
    def body(di_hbm, x_hbm, out_hbm, di_v, rows_v):
        c = lax.axis_index("c")
        s = lax.axis_index("s")
        wid = s * 2 + c

        @pl.loop(0, 20)
        def _win(t):
            w = wid + t * 32

            @pl.when(w < NDWIN)
            def _():
                base = w * DW
                pltpu.sync_copy(di_hbm.at[pl.ds(base, DW)], di_v)
                pltpu.sync_copy(x_hbm.at[di_v], rows_v)
                pltpu.sync_copy(rows_v, out_hbm.at[pl.ds(base, DW)])

    return body(ds_idx, x)


def _stage_c_body(s_ref, dpre_ref, w3_ref, g2_ref, b2_ref, g3_ref, b3_ref,
                  wd_ref, gd_ref, bd_ref, out_ref):
    t0 = _gn_tc(s_ref[0], g2_ref[:, :64], b2_ref[:, :64], G // 2)
    t1 = _gn_tc(s_ref[1], g2_ref[:, 64:], b2_ref[:, 64:], G // 2)
    u = (jnp.dot(t0, w3_ref[:64, :], preferred_element_type=jnp.float32)
         + jnp.dot(t1, w3_ref[64:, :], preferred_element_type=jnp.float32))
    u = _gn_tc(u, g3_ref[...], b3_ref[...], G)
    d = jnp.dot(dpre_ref[...], wd_ref[...], preferred_element_type=jnp.float32)
    d = _gn_tc(d, gd_ref[...], bd_ref[...], G)
    out_ref[...] = u + d


def _stage_c(S, dpre, W3, g2, b2, g3, b3, Wd, gd, bd):
    vec = pl.BlockSpec((1, C), lambda i: (0, 0))
    return pl.pallas_call(
        _stage_c_body,
        grid=(M // BM,),
        in_specs=[
            pl.BlockSpec((2, BM, 64), lambda i: (0, i, 0)),
            pl.BlockSpec((BM, C), lambda i: (i, 0)),
            pl.BlockSpec((C, C), lambda i: (0, 0)),
            vec, vec, vec, vec,
            pl.BlockSpec((C, C), lambda i: (0, 0)),
            vec, vec,
        ],
        out_specs=pl.BlockSpec((BM, C), lambda i: (i, 0)),
        out_shape=jax.ShapeDtypeStruct((M, C), jnp.float32),
    )(S, dpre, W3, g2.reshape(1, C), b2.reshape(1, C), g3.reshape(1, C),
      b3.reshape(1, C), Wd, gd.reshape(1, C), bd.reshape(1, C))


def kernel(x, W1, g1, b1, W2, g2, b2, W3, g3, b3, Wd, gd, bd,
           in_idx, out_idx, koff, ds_idx):
    in_idx = in_idx.astype(jnp.int32)
    out_idx = out_idx.astype(jnp.int32)
    koff = koff.astype(jnp.int32)
    ds_idx = ds_idx.astype(jnp.int32)

    xw = _stage_a(x, W1, g1, b1, W2)
    xw2 = xw.reshape(2 * KN, 64)
    zeros_init = jnp.zeros((MSUB, 64), jnp.float32)
    S = _edge_kernel(in_idx, koff, out_idx, xw2, zeros_init)
    dpre = _ds_kernel(ds_idx, x)
    return _stage_c(S, dpre, W3, g2, b2, g3, b3, Wd, gd, bd)


# speedup vs baseline: 33.4603x; 33.4603x over previous
import functools

import jax
import jax.numpy as jnp
from jax import lax
from jax.experimental import pallas as pl
from jax.experimental.pallas import tpu as pltpu
from jax.experimental.pallas import tpu_sc as plsc

N = 50000
M = 25000
C = 128
E = 400000
K = 27
G = 8
KN = K * N

BN = 400
BM = 200
EW = 128
NWIN = E // EW
MH = 12800
TR = 1024
AR = MH + TR
ASUB = AR // 16
OSUB = MH // 16
DW = 40
NDWIN = M // DW
_EPS = 1e-5


def _gn_tc(h, gamma, beta, groups):
    n, c = h.shape
    cs = c // groups
    gi = lax.broadcasted_iota(jnp.int32, (c, groups), 0) // cs
    gj = lax.broadcasted_iota(jnp.int32, (c, groups), 1)
    Gm = (gi == gj).astype(jnp.float32)
    ti = lax.broadcasted_iota(jnp.int32, (groups, c), 0)
    tj = lax.broadcasted_iota(jnp.int32, (groups, c), 1) // cs
    GmT = (ti == tj).astype(jnp.float32)
    inv_cs = 1.0 / cs
    mu = jnp.dot(h, Gm, preferred_element_type=jnp.float32) * inv_cs
    ex2 = jnp.dot(h * h, Gm, preferred_element_type=jnp.float32) * inv_cs
    var = ex2 - mu * mu
    rstd = lax.rsqrt(var + _EPS)
    mub = jnp.dot(mu, GmT, preferred_element_type=jnp.float32)
    rstdb = jnp.dot(rstd, GmT, preferred_element_type=jnp.float32)
    return (h - mub) * rstdb * gamma + beta


def _stage_a_body(x_ref, w1_ref, g1_ref, b1_ref, w2_ref, xw_ref):
    h = jnp.dot(x_ref[...], w1_ref[...], preferred_element_type=jnp.float32)
    h = _gn_tc(h, g1_ref[...], b1_ref[...], G)
    h = jnp.where(h >= 0, h, 0.01 * h)
    for k in range(K):
        xw_ref[k] = jnp.dot(h, w2_ref[k], preferred_element_type=jnp.float32)


def _stage_a(x, W1, g1, b1, W2):
    return pl.pallas_call(
        _stage_a_body,
        grid=(N // BN,),
        in_specs=[
            pl.BlockSpec((BN, C), lambda i: (i, 0)),
            pl.BlockSpec((C, C), lambda i: (0, 0)),
            pl.BlockSpec((1, C), lambda i: (0, 0)),
            pl.BlockSpec((1, C), lambda i: (0, 0)),
            pl.BlockSpec((K, C, C), lambda i: (0, 0, 0)),
        ],
        out_specs=pl.BlockSpec((K, BN, C), lambda i: (0, i, 0)),
        out_shape=jax.ShapeDtypeStruct((K, N, C), jnp.float32),
    )(x, W1, g1.reshape(1, C), b1.reshape(1, C), W2)


def _edge_kernel(in_idx, koff, out_idx, xw2, zeros_init):
    mesh = plsc.VectorSubcoreMesh(core_axis_name="c", subcore_axis_name="s")

    @functools.partial(
        pl.kernel,
        out_type=jax.ShapeDtypeStruct((2, MH, C), jnp.float32),
        mesh=mesh,
        scratch_types=[
            pltpu.VMEM((EW,), jnp.int32),
            pltpu.VMEM((EW,), jnp.int32),
            pltpu.VMEM((EW,), jnp.int32),
            pltpu.VMEM((EW,), jnp.int32),
            pltpu.VMEM((EW,), jnp.int32),
            pltpu.VMEM((EW, C), jnp.float32),
            pltpu.VMEM_SHARED((AR, C), jnp.float32),
        ],
    )
    def body(ii_hbm, kf_hbm, oi_hbm, xw_hbm, z_hbm, out_hbm,
             ii_v, kf_v, oi_v, gi_v, li_v, rows_v, acc):
        c = lax.axis_index("c")
        s = lax.axis_index("s")
        pltpu.sync_copy(z_hbm, acc.at[pl.ds(s * ASUB, ASUB)])
        plsc.subcore_barrier()
        base_out = c * jnp.int32(MH)

        @pl.loop(0, 196)
        def _win(t):
            w = s + t * 16

            @pl.when(w < NWIN)
            def _():
                base = w * EW
                pltpu.sync_copy(ii_hbm.at[pl.ds(base, EW)], ii_v)
                pltpu.sync_copy(kf_hbm.at[pl.ds(base, EW)], kf_v)
                pltpu.sync_copy(oi_hbm.at[pl.ds(base, EW)], oi_v)
                tbase = jnp.int32(MH) + (w & 7) * jnp.int32(EW)
                lane = lax.iota(jnp.int32, 16)
                for i in range(EW // 16):
                    sl = pl.ds(i * 16, 16)
                    gi_v[sl] = kf_v[sl] * jnp.int32(N) + ii_v[sl]
                    lo = oi_v[sl] - base_out
                    ok = (lo >= 0) & (lo < jnp.int32(MH))
                    trash = lane + (tbase + jnp.int32(i * 16))
                    li_v[sl] = jnp.where(ok, lo, trash)
                pltpu.sync_copy(xw_hbm.at[gi_v], rows_v)
                pltpu.sync_copy(rows_v, acc.at[li_v], add=True)

        plsc.subcore_barrier()
        pltpu.sync_copy(acc.at[pl.ds(s * OSUB, OSUB)],
                        out_hbm.at[c].at[pl.ds(s * OSUB, OSUB)])

    return body(in_idx, koff, out_idx, xw2, zeros_init)


def _ds_kernel(ds_idx, x):
    mesh = plsc.VectorSubcoreMesh(core_axis_name="c", subcore_axis_name="s")

    @functools.partial(
        pl.kernel,
        out_type=jax.ShapeDtypeStruct((M, C), jnp.float32),
        mesh=mesh,
        scratch_types=[
            pltpu.VMEM((DW,), jnp.int32),
            pltpu.VMEM((DW, C), jnp.float32),
        ],
    )
    def body(di_hbm, x_hbm, out_hbm, di_v, rows_v):
        c = lax.axis_index("c")
        s = lax.axis_index("s")
        wid = s * 2 + c

        @pl.loop(0, 20)
        def _win(t):
            w = wid + t * 32

            @pl.when(w < NDWIN)
            def _():
                base = w * DW
                pltpu.sync_copy(di_hbm.at[pl.ds(base, DW)], di_v)
                pltpu.sync_copy(x_hbm.at[di_v], rows_v)
                pltpu.sync_copy(rows_v, out_hbm.at[pl.ds(base, DW)])

    return body(ds_idx, x)


def _stage_c_body(s_ref, dpre_ref, w3_ref, g2_ref, b2_ref, g3_ref, b3_ref,
                  wd_ref, gd_ref, bd_ref, out_ref):
    t = _gn_tc(s_ref[0], g2_ref[...], b2_ref[...], G)
    u = jnp.dot(t, w3_ref[...], preferred_element_type=jnp.float32)
    u = _gn_tc(u, g3_ref[...], b3_ref[...], G)
    d = jnp.dot(dpre_ref[...], wd_ref[...], preferred_element_type=jnp.float32)
    d = _gn_tc(d, gd_ref[...], bd_ref[...], G)
    out_ref[...] = u + d


def _stage_c(S, dpre, W3, g2, b2, g3, b3, Wd, gd, bd):
    vec = pl.BlockSpec((1, C), lambda i: (0, 0))
    nblk = MH // BM
    return pl.pallas_call(
        _stage_c_body,
        grid=(M // BM,),
        in_specs=[
            pl.BlockSpec((1, BM, C), lambda i: (i // nblk, i % nblk, 0)),
            pl.BlockSpec((BM, C), lambda i: (i, 0)),
            pl.BlockSpec((C, C), lambda i: (0, 0)),
            vec, vec, vec, vec,
            pl.BlockSpec((C, C), lambda i: (0, 0)),
            vec, vec,
        ],
        out_specs=pl.BlockSpec((BM, C), lambda i: (i, 0)),
        out_shape=jax.ShapeDtypeStruct((M, C), jnp.float32),
    )(S, dpre, W3, g2.reshape(1, C), b2.reshape(1, C), g3.reshape(1, C),
      b3.reshape(1, C), Wd, gd.reshape(1, C), bd.reshape(1, C))


def kernel(x, W1, g1, b1, W2, g2, b2, W3, g3, b3, Wd, gd, bd,
           in_idx, out_idx, koff, ds_idx):
    in_idx = in_idx.astype(jnp.int32)
    out_idx = out_idx.astype(jnp.int32)
    koff = koff.astype(jnp.int32)
    ds_idx = ds_idx.astype(jnp.int32)

    xw = _stage_a(x, W1, g1, b1, W2)
    xw2 = xw.reshape(KN, C)
    zeros_init = jnp.zeros((ASUB, C), jnp.float32)
    S = _edge_kernel(in_idx, koff, out_idx, xw2, zeros_init)
    dpre = _ds_kernel(ds_idx, x)
    return _stage_c(S, dpre, W3, g2, b2, g3, b3, Wd, gd, bd)


# speedup vs baseline: 33.9286x; 1.0140x over previous
import functools

import jax
import jax.numpy as jnp
from jax import lax
from jax.experimental import pallas as pl
from jax.experimental.pallas import tpu as pltpu
from jax.experimental.pallas import tpu_sc as plsc

N = 50000
M = 25000
C = 128
E = 400000
K = 27
G = 8
KN = K * N

BN = 400
BM = 200
EW = 96
NWINP = 4192
EP = NWINP * EW
BW = 32
MH = 12800
TR = 192
AR = MH + TR
ASUB = AR // 16
OSUB = MH // 16
OPAD = 2 * MH
DW = 40
NDWIN = M // DW
_EPS = 1e-5


def _gn_tc(h, gamma, beta, groups):
    n, c = h.shape
    cs = c // groups
    gi = lax.broadcasted_iota(jnp.int32, (c, groups), 0) // cs
    gj = lax.broadcasted_iota(jnp.int32, (c, groups), 1)
    Gm = (gi == gj).astype(jnp.float32)
    ti = lax.broadcasted_iota(jnp.int32, (groups, c), 0)
    tj = lax.broadcasted_iota(jnp.int32, (groups, c), 1) // cs
    GmT = (ti == tj).astype(jnp.float32)
    inv_cs = 1.0 / cs
    mu = jnp.dot(h, Gm, preferred_element_type=jnp.float32) * inv_cs
    ex2 = jnp.dot(h * h, Gm, preferred_element_type=jnp.float32) * inv_cs
    var = ex2 - mu * mu
    rstd = lax.rsqrt(var + _EPS)
    mub = jnp.dot(mu, GmT, preferred_element_type=jnp.float32)
    rstdb = jnp.dot(rstd, GmT, preferred_element_type=jnp.float32)
    return (h - mub) * rstdb * gamma + beta


def _stage_a_body(x_ref, w1_ref, g1_ref, b1_ref, w2_ref, xw_ref):
    h = jnp.dot(x_ref[...], w1_ref[...], preferred_element_type=jnp.float32)
    h = _gn_tc(h, g1_ref[...], b1_ref[...], G)
    h = jnp.where(h >= 0, h, 0.01 * h)
    for k in range(K):
        xw_ref[k] = jnp.dot(h, w2_ref[k], preferred_element_type=jnp.float32)


def _stage_a(x, W1, g1, b1, W2):
    return pl.pallas_call(
        _stage_a_body,
        grid=(N // BN,),
        in_specs=[
            pl.BlockSpec((BN, C), lambda i: (i, 0)),
            pl.BlockSpec((C, C), lambda i: (0, 0)),
            pl.BlockSpec((1, C), lambda i: (0, 0)),
            pl.BlockSpec((1, C), lambda i: (0, 0)),
            pl.BlockSpec((K, C, C), lambda i: (0, 0, 0)),
        ],
        out_specs=pl.BlockSpec((K, BN, C), lambda i: (0, i, 0)),
        out_shape=jax.ShapeDtypeStruct((K, N, C), jnp.float32),
    )(x, W1, g1.reshape(1, C), b1.reshape(1, C), W2)


def _stage_p_body(ii_ref, kf_ref, oi_ref, p_ref):
    i = pl.program_id(0)
    ii = ii_ref[...]
    kf = kf_ref[...]
    oi = oi_ref[...]
    w = i * BW + lax.broadcasted_iota(jnp.int32, (BW, EW), 0)
    lane = lax.broadcasted_iota(jnp.int32, (BW, EW), 1)
    gidx = kf * jnp.int32(N) + ii
    trash = jnp.int32(MH) + (w & 1) * jnp.int32(EW) + lane
    li0 = jnp.where(oi < MH, oi, trash)
    lo1 = oi - jnp.int32(MH)
    li1 = jnp.where((lo1 >= 0) & (lo1 < MH), lo1, trash)
    p_ref[0, :, 0, :] = gidx
    p_ref[0, :, 1, :] = li0
    p_ref[1, :, 0, :] = gidx
    p_ref[1, :, 1, :] = li1


def _stage_p(in_idx, koff, out_idx):
    ii = jnp.pad(in_idx, (0, EP - E)).reshape(NWINP, EW)
    kf = jnp.pad(koff, (0, EP - E)).reshape(NWINP, EW)
    oi = jnp.pad(out_idx, (0, EP - E),
                 constant_values=OPAD).reshape(NWINP, EW)
    blk = pl.BlockSpec((BW, EW), lambda i: (i, 0))
    return pl.pallas_call(
        _stage_p_body,
        grid=(NWINP // BW,),
        in_specs=[blk, blk, blk],
        out_specs=pl.BlockSpec((2, BW, 2, EW), lambda i: (0, i, 0, 0)),
        out_shape=jax.ShapeDtypeStruct((2, NWINP, 2, EW), jnp.int32),
    )(ii, kf, oi)


def _edge_kernel(P, xw2, zeros_init):
    mesh = plsc.VectorSubcoreMesh(core_axis_name="c", subcore_axis_name="s")
    NB = 2

    @functools.partial(
        pl.kernel,
        out_type=jax.ShapeDtypeStruct((2, MH, C), jnp.float32),
        mesh=mesh,
        scratch_types=[
            pltpu.VMEM((NB, 2, EW), jnp.int32),
            pltpu.VMEM((NB, EW, C), jnp.float32),
            pltpu.VMEM_SHARED((AR, C), jnp.float32),
        ] + [pltpu.SemaphoreType.DMA] * (3 * NB),
    )
    def body(p_hbm, xw_hbm, z_hbm, out_hbm, pb, rows, acc, *sems):
        isem = sems[:NB]
        gsem = sems[NB:2 * NB]
        ssem = sems[2 * NB:]
        c = lax.axis_index("c")
        s = lax.axis_index("s")
        pltpu.sync_copy(z_hbm, acc.at[pl.ds(s * ASUB, ASUB)])
        plsc.subcore_barrier()
        pc_hbm = p_hbm.at[c]

        @pl.loop(0, 262 // NB)
        def _quad(u):
            w0 = s + (u * NB) * 16
            for j in range(NB):
                pltpu.async_copy(pc_hbm.at[w0 + j * 16], pb.at[j], isem[j])
            gets = []
            for j in range(NB):
                pltpu.make_async_copy(pc_hbm.at[w0], pb.at[j], isem[j]).wait()
                gets.append(pltpu.async_copy(
                    xw_hbm.at[pb.at[j, 0]], rows.at[j], gsem[j]))
            for j in range(NB):
                gets[j].wait()
                pltpu.async_copy(rows.at[j], acc.at[pb.at[j, 1]], ssem[j],
                                 add=True)
            for j in range(NB):
                pltpu.make_async_copy(rows.at[j], acc.at[pb.at[j, 1]],
                                      ssem[j]).wait()

        plsc.subcore_barrier()
        pltpu.sync_copy(acc.at[pl.ds(s * OSUB, OSUB)],
                        out_hbm.at[c].at[pl.ds(s * OSUB, OSUB)])

    return body(P, xw2, zeros_init)


def _ds_kernel(ds_idx, x):
    mesh = plsc.VectorSubcoreMesh(core_axis_name="c", subcore_axis_name="s")

    @functools.partial(
        pl.kernel,
        out_type=jax.ShapeDtypeStruct((M, C), jnp.float32),
        mesh=mesh,
        scratch_types=[
            pltpu.VMEM((DW,), jnp.int32),
            pltpu.VMEM((DW, C), jnp.float32),
        ],
    )
    def body(di_hbm, x_hbm, out_hbm, di_v, rows_v):
        c = lax.axis_index("c")
        s = lax.axis_index("s")
        wid = s * 2 + c

        @pl.loop(0, 20)
        def _win(t):
            w = wid + t * 32

            @pl.when(w < NDWIN)
            def _():
                base = w * DW
                pltpu.sync_copy(di_hbm.at[pl.ds(base, DW)], di_v)
                pltpu.sync_copy(x_hbm.at[di_v], rows_v)
                pltpu.sync_copy(rows_v, out_hbm.at[pl.ds(base, DW)])

    return body(ds_idx, x)


def _stage_c_body(s_ref, dpre_ref, w3_ref, g2_ref, b2_ref, g3_ref, b3_ref,
                  wd_ref, gd_ref, bd_ref, out_ref):
    t = _gn_tc(s_ref[0], g2_ref[...], b2_ref[...], G)
    u = jnp.dot(t, w3_ref[...], preferred_element_type=jnp.float32)
    u = _gn_tc(u, g3_ref[...], b3_ref[...], G)
    d = jnp.dot(dpre_ref[...], wd_ref[...], preferred_element_type=jnp.float32)
    d = _gn_tc(d, gd_ref[...], bd_ref[...], G)
    out_ref[...] = u + d


def _stage_c(S, dpre, W3, g2, b2, g3, b3, Wd, gd, bd):
    vec = pl.BlockSpec((1, C), lambda i: (0, 0))
    nblk = MH // BM
    return pl.pallas_call(
        _stage_c_body,
        grid=(M // BM,),
        in_specs=[
            pl.BlockSpec((1, BM, C), lambda i: (i // nblk, i % nblk, 0)),
            pl.BlockSpec((BM, C), lambda i: (i, 0)),
            pl.BlockSpec((C, C), lambda i: (0, 0)),
            vec, vec, vec, vec,
            pl.BlockSpec((C, C), lambda i: (0, 0)),
            vec, vec,
        ],
        out_specs=pl.BlockSpec((BM, C), lambda i: (i, 0)),
        out_shape=jax.ShapeDtypeStruct((M, C), jnp.float32),
    )(S, dpre, W3, g2.reshape(1, C), b2.reshape(1, C), g3.reshape(1, C),
      b3.reshape(1, C), Wd, gd.reshape(1, C), bd.reshape(1, C))


def kernel(x, W1, g1, b1, W2, g2, b2, W3, g3, b3, Wd, gd, bd,
           in_idx, out_idx, koff, ds_idx):
    in_idx = in_idx.astype(jnp.int32)
    out_idx = out_idx.astype(jnp.int32)
    koff = koff.astype(jnp.int32)
    ds_idx = ds_idx.astype(jnp.int32)

    xw = _stage_a(x, W1, g1, b1, W2)
    xw2 = xw.reshape(KN, C)
    P = _stage_p(in_idx, koff, out_idx)
    zeros_init = jnp.zeros((ASUB, C), jnp.float32)
    S = _edge_kernel(P, xw2, zeros_init)
    dpre = _ds_kernel(ds_idx, x)
    return _stage_c(S, dpre, W3, g2, b2, g3, b3, Wd, gd, bd)


# speedup vs baseline: 36.8718x; 1.0867x over previous
import functools

import jax
import jax.numpy as jnp
from jax import lax
from jax.experimental import pallas as pl
from jax.experimental.pallas import tpu as pltpu
from jax.experimental.pallas import tpu_sc as plsc

N = 50000
M = 25000
C = 128
E = 400000
K = 27
G = 8
KN = K * N

BN = 400
BM = 1000
EW = 96
NWINP = 4192
EP = NWINP * EW
BW = 32
MH = 12800
TR = 192
AR = MH + TR
ASUB = AR // 16
OSUB = MH // 16
OPAD = 2 * MH
DW = 40
NDWIN = M // DW
_EPS = 1e-5


def _gn_tc(h, gamma, beta, groups):
    n, c = h.shape
    cs = c // groups
    gi = lax.broadcasted_iota(jnp.int32, (c, groups), 0) // cs
    gj = lax.broadcasted_iota(jnp.int32, (c, groups), 1)
    Gm = (gi == gj).astype(jnp.float32)
    ti = lax.broadcasted_iota(jnp.int32, (groups, c), 0)
    tj = lax.broadcasted_iota(jnp.int32, (groups, c), 1) // cs
    GmT = (ti == tj).astype(jnp.float32)
    inv_cs = 1.0 / cs
    mu = jnp.dot(h, Gm, preferred_element_type=jnp.float32) * inv_cs
    ex2 = jnp.dot(h * h, Gm, preferred_element_type=jnp.float32) * inv_cs
    var = ex2 - mu * mu
    rstd = lax.rsqrt(var + _EPS)
    mub = jnp.dot(mu, GmT, preferred_element_type=jnp.float32)
    rstdb = jnp.dot(rstd, GmT, preferred_element_type=jnp.float32)
    return (h - mub) * rstdb * gamma + beta


def _stage_a_body(x_ref, w1_ref, g1_ref, b1_ref, w2_ref, xw_ref):
    h = jnp.dot(x_ref[...], w1_ref[...], preferred_element_type=jnp.float32)
    h = _gn_tc(h, g1_ref[...], b1_ref[...], G)
    h = jnp.where(h >= 0, h, 0.01 * h)
    for k in range(K):
        xw_ref[k] = jnp.dot(h, w2_ref[k], preferred_element_type=jnp.float32)


def _stage_a(x, W1, g1, b1, W2):
    return pl.pallas_call(
        _stage_a_body,
        grid=(N // BN,),
        in_specs=[
            pl.BlockSpec((BN, C), lambda i: (i, 0)),
            pl.BlockSpec((C, C), lambda i: (0, 0)),
            pl.BlockSpec((1, C), lambda i: (0, 0)),
            pl.BlockSpec((1, C), lambda i: (0, 0)),
            pl.BlockSpec((K, C, C), lambda i: (0, 0, 0)),
        ],
        out_specs=pl.BlockSpec((K, BN, C), lambda i: (0, i, 0)),
        out_shape=jax.ShapeDtypeStruct((K, N, C), jnp.float32),
    )(x, W1, g1.reshape(1, C), b1.reshape(1, C), W2)


def _stage_p_body(ii_ref, kf_ref, oi_ref, p_ref):
    i = pl.program_id(0)
    ii = ii_ref[...]
    kf = kf_ref[...]
    oi = oi_ref[...]
    w = i * BW + lax.broadcasted_iota(jnp.int32, (BW, EW), 0)
    lane = lax.broadcasted_iota(jnp.int32, (BW, EW), 1)
    gidx = kf * jnp.int32(N) + ii
    trash = jnp.int32(MH) + (w & 1) * jnp.int32(EW) + lane
    li0 = jnp.where(oi < MH, oi, trash)
    lo1 = oi - jnp.int32(MH)
    li1 = jnp.where((lo1 >= 0) & (lo1 < MH), lo1, trash)
    p_ref[0, :, 0, :] = gidx
    p_ref[0, :, 1, :] = li0
    p_ref[1, :, 0, :] = gidx
    p_ref[1, :, 1, :] = li1


def _stage_p(in_idx, koff, out_idx):
    ii = jnp.pad(in_idx, (0, EP - E)).reshape(NWINP, EW)
    kf = jnp.pad(koff, (0, EP - E)).reshape(NWINP, EW)
    oi = jnp.pad(out_idx, (0, EP - E),
                 constant_values=OPAD).reshape(NWINP, EW)
    blk = pl.BlockSpec((BW, EW), lambda i: (i, 0))
    return pl.pallas_call(
        _stage_p_body,
        grid=(NWINP // BW,),
        in_specs=[blk, blk, blk],
        out_specs=pl.BlockSpec((2, BW, 2, EW), lambda i: (0, i, 0, 0)),
        out_shape=jax.ShapeDtypeStruct((2, NWINP, 2, EW), jnp.int32),
    )(ii, kf, oi)


def _edge_kernel(P, xw2, zeros_init):
    mesh = plsc.VectorSubcoreMesh(core_axis_name="c", subcore_axis_name="s")
    NB = 2

    @functools.partial(
        pl.kernel,
        out_type=jax.ShapeDtypeStruct((2, MH, C), jnp.float32),
        mesh=mesh,
        scratch_types=[
            pltpu.VMEM((NB, 2, EW), jnp.int32),
            pltpu.VMEM((NB, EW, C), jnp.float32),
            pltpu.VMEM_SHARED((AR, C), jnp.float32),
        ] + [pltpu.SemaphoreType.DMA] * (3 * NB),
    )
    def body(p_hbm, xw_hbm, z_hbm, out_hbm, pb, rows, acc, *sems):
        isem = sems[:NB]
        gsem = sems[NB:2 * NB]
        ssem = sems[2 * NB:]
        c = lax.axis_index("c")
        s = lax.axis_index("s")
        pltpu.sync_copy(z_hbm, acc.at[pl.ds(s * ASUB, ASUB)])
        plsc.subcore_barrier()
        pc_hbm = p_hbm.at[c]

        @pl.loop(0, 262 // NB)
        def _quad(u):
            w0 = s + (u * NB) * 16
            for j in range(NB):
                @pl.when(u > 0)
                def _():
                    pltpu.make_async_copy(rows.at[j], acc.at[pb.at[j, 1]],
                                          ssem[j]).wait()
                pltpu.async_copy(pc_hbm.at[w0 + j * 16], pb.at[j], isem[j])
            gets = []
            for j in range(NB):
                pltpu.make_async_copy(pc_hbm.at[w0], pb.at[j], isem[j]).wait()
                gets.append(pltpu.async_copy(
                    xw_hbm.at[pb.at[j, 0]], rows.at[j], gsem[j]))
            for j in range(NB):
                gets[j].wait()
                pltpu.async_copy(rows.at[j], acc.at[pb.at[j, 1]], ssem[j],
                                 add=True)

        for j in range(NB):
            pltpu.make_async_copy(rows.at[j], acc.at[pb.at[j, 1]],
                                  ssem[j]).wait()

        plsc.subcore_barrier()
        pltpu.sync_copy(acc.at[pl.ds(s * OSUB, OSUB)],
                        out_hbm.at[c].at[pl.ds(s * OSUB, OSUB)])

    return body(P, xw2, zeros_init)


def _ds_kernel(ds_idx, x):
    mesh = plsc.VectorSubcoreMesh(core_axis_name="c", subcore_axis_name="s")

    @functools.partial(
        pl.kernel,
        out_type=jax.ShapeDtypeStruct((M, C), jnp.float32),
        mesh=mesh,
        scratch_types=[
            pltpu.VMEM((DW,), jnp.int32),
            pltpu.VMEM((DW, C), jnp.float32),
        ],
    )
    def body(di_hbm, x_hbm, out_hbm, di_v, rows_v):
        c = lax.axis_index("c")
        s = lax.axis_index("s")
        wid = s * 2 + c

        @pl.loop(0, 20)
        def _win(t):
            w = wid + t * 32

            @pl.when(w < NDWIN)
            def _():
                base = w * DW
                pltpu.sync_copy(di_hbm.at[pl.ds(base, DW)], di_v)
                pltpu.sync_copy(x_hbm.at[di_v], rows_v)
                pltpu.sync_copy(rows_v, out_hbm.at[pl.ds(base, DW)])

    return body(ds_idx, x)


def _stage_c_body(s_ref, dpre_ref, w3_ref, g2_ref, b2_ref, g3_ref, b3_ref,
                  wd_ref, gd_ref, bd_ref, out_ref):
    t = _gn_tc(s_ref[...], g2_ref[...], b2_ref[...], G)
    u = jnp.dot(t, w3_ref[...], preferred_element_type=jnp.float32)
    u = _gn_tc(u, g3_ref[...], b3_ref[...], G)
    d = jnp.dot(dpre_ref[...], wd_ref[...], preferred_element_type=jnp.float32)
    d = _gn_tc(d, gd_ref[...], bd_ref[...], G)
    out_ref[...] = u + d


def _stage_c(S, dpre, W3, g2, b2, g3, b3, Wd, gd, bd):
    S = S.reshape(2 * MH, C)
    vec = pl.BlockSpec((1, C), lambda i: (0, 0))
    return pl.pallas_call(
        _stage_c_body,
        grid=(M // BM,),
        in_specs=[
            pl.BlockSpec((BM, C), lambda i: (i, 0)),
            pl.BlockSpec((BM, C), lambda i: (i, 0)),
            pl.BlockSpec((C, C), lambda i: (0, 0)),
            vec, vec, vec, vec,
            pl.BlockSpec((C, C), lambda i: (0, 0)),
            vec, vec,
        ],
        out_specs=pl.BlockSpec((BM, C), lambda i: (i, 0)),
        out_shape=jax.ShapeDtypeStruct((M, C), jnp.float32),
    )(S, dpre, W3, g2.reshape(1, C), b2.reshape(1, C), g3.reshape(1, C),
      b3.reshape(1, C), Wd, gd.reshape(1, C), bd.reshape(1, C))


def kernel(x, W1, g1, b1, W2, g2, b2, W3, g3, b3, Wd, gd, bd,
           in_idx, out_idx, koff, ds_idx):
    in_idx = in_idx.astype(jnp.int32)
    out_idx = out_idx.astype(jnp.int32)
    koff = koff.astype(jnp.int32)
    ds_idx = ds_idx.astype(jnp.int32)

    xw = _stage_a(x, W1, g1, b1, W2)
    xw2 = xw.reshape(KN, C)
    P = _stage_p(in_idx, koff, out_idx)
    zeros_init = jnp.zeros((ASUB, C), jnp.float32)
    S = _edge_kernel(P, xw2, zeros_init)
    dpre = _ds_kernel(ds_idx, x)
    return _stage_c(S, dpre, W3, g2, b2, g3, b3, Wd, gd, bd)


# speedup vs baseline: 58.8581x; 1.5963x over previous
import dataclasses
import functools

import jax
import jax.numpy as jnp
from jax import lax
from jax.experimental import pallas as pl
from jax.experimental.pallas import tpu as pltpu
from jax.experimental.pallas import tpu_sc as plsc

N = 50000
M = 25000
C = 128
E = 400000
K = 27
G = 8
KN = K * N

BN = 400
BM = 1000
EW = 128
NWINP = 3136
EP = NWINP * EW
BW = 112
EB = BW * EW
EW2 = 64
NWIN2 = EP // EW2
TSUB = NWIN2 // 16
MH = 12800
AR = MH + 128
ASUB = AR // 16
OSUB = MH // 16
DW = 40
NDWIN = M // DW
NB = 2
_EPS = 1e-5

_SC_MESH = dict(core_axis_name="c", subcore_axis_name="s")


def _sc_params():
    cp = pltpu.CompilerParams()
    return dataclasses.replace(cp, needs_layout_passes=False,
                               use_tc_tiling_on_sc=False)


def _gn_tc(h, gamma, beta, groups):
    n, c = h.shape
    cs = c // groups
    gi = lax.broadcasted_iota(jnp.int32, (c, groups), 0) // cs
    gj = lax.broadcasted_iota(jnp.int32, (c, groups), 1)
    Gm = (gi == gj).astype(jnp.float32)
    ti = lax.broadcasted_iota(jnp.int32, (groups, c), 0)
    tj = lax.broadcasted_iota(jnp.int32, (groups, c), 1) // cs
    GmT = (ti == tj).astype(jnp.float32)
    inv_cs = 1.0 / cs
    mu = jnp.dot(h, Gm, preferred_element_type=jnp.float32) * inv_cs
    ex2 = jnp.dot(h * h, Gm, preferred_element_type=jnp.float32) * inv_cs
    var = ex2 - mu * mu
    rstd = lax.rsqrt(var + _EPS)
    mub = jnp.dot(mu, GmT, preferred_element_type=jnp.float32)
    rstdb = jnp.dot(rstd, GmT, preferred_element_type=jnp.float32)
    return (h - mub) * rstdb * gamma + beta


def _stage_a_body(x_ref, w1_ref, g1_ref, b1_ref, w2_ref, xw_ref):
    h = jnp.dot(x_ref[...], w1_ref[...], preferred_element_type=jnp.float32)
    h = _gn_tc(h, g1_ref[...], b1_ref[...], G)
    h = jnp.where(h >= 0, h, 0.01 * h)
    for k in range(K):
        xw_ref[k] = jnp.dot(h, w2_ref[k], preferred_element_type=jnp.float32)


def _stage_a(x, W1, g1, b1, W2):
    return pl.pallas_call(
        _stage_a_body,
        grid=(N // BN,),
        in_specs=[
            pl.BlockSpec((BN, C), lambda i: (i, 0)),
            pl.BlockSpec((C, C), lambda i: (0, 0)),
            pl.BlockSpec((1, C), lambda i: (0, 0)),
            pl.BlockSpec((1, C), lambda i: (0, 0)),
            pl.BlockSpec((K, C, C), lambda i: (0, 0, 0)),
        ],
        out_specs=pl.BlockSpec((K, BN, C), lambda i: (0, i, 0)),
        out_shape=jax.ShapeDtypeStruct((K, N, C), jnp.float32),
    )(x, W1, g1.reshape(1, C), b1.reshape(1, C), W2)


def _tri_lanes(n):
    a = lax.broadcasted_iota(jnp.int32, (n, n), 0)
    b = lax.broadcasted_iota(jnp.int32, (n, n), 1)
    return (a < b).astype(jnp.float32)


def _tri_rows(n):
    a = lax.broadcasted_iota(jnp.int32, (n, n), 0)
    b = lax.broadcasted_iota(jnp.int32, (n, n), 1)
    return (b < a).astype(jnp.float32)


def _stage_p_body(ii_ref, kf_ref, oi_ref, p_ref, cnt_ref, carry):
    i = pl.program_id(0)

    @pl.when(i == 0)
    def _():
        carry[0] = 0
        carry[1] = 0

    ii = ii_ref[...]
    kf = kf_ref[...]
    oi = oi_ref[...]
    row = lax.broadcasted_iota(jnp.int32, (BW, EW), 0)
    lane = lax.broadcasted_iota(jnp.int32, (BW, EW), 1)
    eidx = i * EB + row * EW + lane
    is_pad = eidx >= E
    gidx = jnp.where(is_pad, lane, kf * jnp.int32(N) + ii)

    c0 = carry[0]
    c1 = carry[1]
    b1r = (oi >= MH).astype(jnp.int32)
    s0_real = jnp.sum(jnp.where(is_pad, 0, 1 - b1r))
    n0_real_total = c0 + s0_real
    padn0 = (-n0_real_total) % jnp.int32(EW)
    pad_rank = eidx - jnp.int32(E)
    b1pad = (pad_rank >= padn0).astype(jnp.int32)
    b1i = jnp.where(is_pad, b1pad, b1r)

    trash = jnp.int32(MH) + lane
    li = jnp.where(is_pad, trash,
                   jnp.where(b1i == 1, oi - jnp.int32(MH), oi))

    b1f = b1i.astype(jnp.float32)
    excl1 = jnp.dot(b1f, _tri_lanes(EW), preferred_element_type=jnp.float32)
    rs1 = jnp.dot(b1f, jnp.ones((EW, 1), jnp.float32),
                  preferred_element_type=jnp.float32)
    rp1 = jnp.dot(_tri_rows(BW), rs1, preferred_element_type=jnp.float32)
    pos1 = (rp1 + excl1).astype(jnp.int32)
    iflat = row * EW + lane
    pos0 = iflat - pos1
    dst = jnp.where(b1i == 1, jnp.int32(EP - 1) - (c1 + pos1), c0 + pos0)

    p_ref[0] = gidx
    p_ref[1] = li
    p_ref[2] = dst

    s1 = jnp.sum(b1i)
    s0 = jnp.int32(EB) - s1
    carry[0] = c0 + s0
    carry[1] = c1 + s1

    @pl.when(i == (NWINP // BW) - 1)
    def _():
        nwin0 = (c0 + s0) // jnp.int32(EW2)
        l2 = lax.broadcasted_iota(jnp.int32, (1, C), 1)
        cnt_ref[...] = jnp.where(
            l2 == 0, nwin0,
            jnp.where(l2 == 1, jnp.int32(NWIN2) - nwin0, 0))


def _stage_p(in_idx, koff, out_idx):
    ii = jnp.pad(in_idx, (0, EP - E)).reshape(NWINP, EW)
    kf = jnp.pad(koff, (0, EP - E)).reshape(NWINP, EW)
    oi = jnp.pad(out_idx, (0, EP - E)).reshape(NWINP, EW)
    blk = pl.BlockSpec((BW, EW), lambda i: (i, 0))
    return pl.pallas_call(
        _stage_p_body,
        grid=(NWINP // BW,),
        in_specs=[blk, blk, blk],
        out_specs=[
            pl.BlockSpec((3, BW, EW), lambda i: (0, i, 0)),
            pl.BlockSpec((1, C), lambda i: (0, 0)),
        ],
        out_shape=[
            jax.ShapeDtypeStruct((3, NWINP, EW), jnp.int32),
            jax.ShapeDtypeStruct((1, C), jnp.int32),
        ],
        scratch_shapes=[pltpu.SMEM((2,), jnp.int32)],
    )(ii, kf, oi)


def _reorder_kernel(P):
    mesh = plsc.VectorSubcoreMesh(**_SC_MESH)

    @functools.partial(
        pl.kernel,
        out_type=jax.ShapeDtypeStruct((EP, 16), jnp.int32),
        mesh=mesh,
        compiler_params=_sc_params(),
        scratch_types=[
            pltpu.VMEM((NB, EW), jnp.int32),
            pltpu.VMEM((NB, EW), jnp.int32),
            pltpu.VMEM((NB, EW), jnp.int32),
            pltpu.VMEM((NB, EW, 16), jnp.int32),
        ] + [pltpu.SemaphoreType.DMA] * (2 * NB),
    )
    def body(p_hbm, o_hbm, gi_v, li_v, di_v, pe, *sems):
        isem = sems[:NB]
        ssem = sems[NB:]
        c = lax.axis_index("c")
        s = lax.axis_index("s")
        wid = s * 2 + c
        rows16 = lax.iota(jnp.int32, 16)
        z16 = jnp.zeros((16,), jnp.int32)

        @pl.loop(0, NWINP // 32 // NB)
        def _it(u):
            for j in range(NB):
                @pl.when(u > 0)
                def _():
                    pltpu.make_async_copy(pe.at[j], o_hbm.at[di_v.at[j]],
                                          ssem[j]).wait()
                w = wid + (u * NB + j) * 32
                pltpu.async_copy(p_hbm.at[0].at[w], gi_v.at[j], isem[j])
                pltpu.async_copy(p_hbm.at[1].at[w], li_v.at[j], isem[j])
                pltpu.async_copy(p_hbm.at[2].at[w], di_v.at[j], isem[j])
            for j in range(NB):
                w = wid + (u * NB + j) * 32
                pltpu.make_async_copy(p_hbm.at[0].at[w],
                                      gi_v.at[j], isem[j]).wait()
                pltpu.make_async_copy(p_hbm.at[1].at[w],
                                      li_v.at[j], isem[j]).wait()
                pltpu.make_async_copy(p_hbm.at[2].at[w],
                                      di_v.at[j], isem[j]).wait()
                for t in range(EW // 16):
                    sl = pl.ds(t * 16, 16)
                    plsc.store_scatter(pe.at[j], [rows16 + t * 16, z16],
                                       gi_v[j, sl])
                    plsc.store_scatter(pe.at[j], [rows16 + t * 16, z16 + 1],
                                       li_v[j, sl])
                pltpu.async_copy(pe.at[j], o_hbm.at[di_v.at[j]], ssem[j])

        for j in range(NB):
            pltpu.make_async_copy(pe.at[j], o_hbm.at[di_v.at[j]],
                                  ssem[j]).wait()

    return body(P)


def _edge_kernel(Ppart, counts, xw2, zeros_init):
    mesh = plsc.VectorSubcoreMesh(**_SC_MESH)
    TSUB = NWIN2 // 16

    @functools.partial(
        pl.kernel,
        out_type=jax.ShapeDtypeStruct((2, MH, C), jnp.float32),
        mesh=mesh,
        compiler_params=_sc_params(),
        scratch_types=[
            pltpu.VMEM((16,), jnp.int32),
            pltpu.VMEM((NB, EW2, 16), jnp.int32),
            pltpu.VMEM((NB, EW2), jnp.int32),
            pltpu.VMEM((NB, EW2), jnp.int32),
            pltpu.VMEM((NB, EW2, C), jnp.float32),
            pltpu.VMEM_SHARED((AR, C), jnp.float32),
        ] + [pltpu.SemaphoreType.DMA] * (3 * NB),
    )
    def body(pp_hbm, cnt_hbm, xw_hbm, z_hbm, out_hbm,
             cnt_v, pb, gi_v, li_v, rows, acc, *sems):
        isem = sems[:NB]
        gsem = sems[NB:2 * NB]
        ssem = sems[2 * NB:]
        c = lax.axis_index("c")
        s = lax.axis_index("s")
        pltpu.sync_copy(z_hbm, acc.at[pl.ds(s * ASUB, ASUB)])
        pltpu.sync_copy(cnt_hbm.at[0].at[pl.ds(0, 16)], cnt_v)
        lane16 = lax.iota(jnp.int32, 16)
        nwin = jnp.sum(jnp.where(lane16 == c, cnt_v[...], 0))
        plsc.subcore_barrier()
        rows16 = lax.iota(jnp.int32, 16)
        z16 = jnp.zeros((16,), jnp.int32)

        @pl.loop(0, TSUB // NB)
        def _it(u):
            for j in range(NB):
                w = s + (u * NB + j) * 16

                @pl.when(w < nwin)
                def _():
                    @pl.when(u > 0)
                    def _():
                        pltpu.make_async_copy(
                            rows.at[j], acc.at[li_v.at[j]], ssem[j]).wait()
                    base = jnp.where(c == 0, w * EW2,
                                     jnp.int32(EP) - (w + 1) * EW2)
                    pltpu.async_copy(pp_hbm.at[pl.ds(base, EW2)], pb.at[j],
                                     isem[j])
            for j in range(NB):
                w = s + (u * NB + j) * 16

                @pl.when(w < nwin)
                def _():
                    pltpu.make_async_copy(pp_hbm.at[pl.ds(0, EW2)], pb.at[j],
                                          isem[j]).wait()
                    for t in range(EW2 // 16):
                        sl = pl.ds(t * 16, 16)
                        gi_v[j, sl] = plsc.load_gather(
                            pb.at[j], [rows16 + t * 16, z16])
                        li_v[j, sl] = plsc.load_gather(
                            pb.at[j], [rows16 + t * 16, z16 + 1])
                    pltpu.async_copy(xw_hbm.at[gi_v.at[j]], rows.at[j],
                                     gsem[j])
            for j in range(NB):
                w = s + (u * NB + j) * 16

                @pl.when(w < nwin)
                def _():
                    pltpu.make_async_copy(xw_hbm.at[gi_v.at[j]], rows.at[j],
                                          gsem[j]).wait()
                    pltpu.async_copy(rows.at[j], acc.at[li_v.at[j]], ssem[j],
                                     add=True)

        for j in range(NB):
            @pl.when((s + j * 16) < nwin)
            def _():
                pltpu.make_async_copy(rows.at[j], acc.at[li_v.at[j]],
                                      ssem[j]).wait()

        plsc.subcore_barrier()
        pltpu.sync_copy(acc.at[pl.ds(s * OSUB, OSUB)],
                        out_hbm.at[c].at[pl.ds(s * OSUB, OSUB)])

    return body(Ppart, counts, xw2, zeros_init)


def _ds_kernel(ds_idx, x):
    mesh = plsc.VectorSubcoreMesh(**_SC_MESH)

    @functools.partial(
        pl.kernel,
        out_type=jax.ShapeDtypeStruct((M, C), jnp.float32),
        mesh=mesh,
        scratch_types=[
            pltpu.VMEM((DW,), jnp.int32),
            pltpu.VMEM((DW, C), jnp.float32),
        ],
    )
    def body(di_hbm, x_hbm, out_hbm, di_v, rows_v):
        c = lax.axis_index("c")
        s = lax.axis_index("s")
        wid = s * 2 + c

        @pl.loop(0, 20)
        def _win(t):
            w = wid + t * 32

            @pl.when(w < NDWIN)
            def _():
                base = w * DW
                pltpu.sync_copy(di_hbm.at[pl.ds(base, DW)], di_v)
                pltpu.sync_copy(x_hbm.at[di_v], rows_v)
                pltpu.sync_copy(rows_v, out_hbm.at[pl.ds(base, DW)])

    return body(ds_idx, x)


def _stage_c_body(s_ref, dpre_ref, w3_ref, g2_ref, b2_ref, g3_ref, b3_ref,
                  wd_ref, gd_ref, bd_ref, out_ref):
    t = _gn_tc(s_ref[...], g2_ref[...], b2_ref[...], G)
    u = jnp.dot(t, w3_ref[...], preferred_element_type=jnp.float32)
    u = _gn_tc(u, g3_ref[...], b3_ref[...], G)
    d = jnp.dot(dpre_ref[...], wd_ref[...], preferred_element_type=jnp.float32)
    d = _gn_tc(d, gd_ref[...], bd_ref[...], G)
    out_ref[...] = u + d


def _stage_c(S, dpre, W3, g2, b2, g3, b3, Wd, gd, bd):
    S = S.reshape(2 * MH, C)
    vec = pl.BlockSpec((1, C), lambda i: (0, 0))
    return pl.pallas_call(
        _stage_c_body,
        grid=(M // BM,),
        in_specs=[
            pl.BlockSpec((BM, C), lambda i: (i, 0)),
            pl.BlockSpec((BM, C), lambda i: (i, 0)),
            pl.BlockSpec((C, C), lambda i: (0, 0)),
            vec, vec, vec, vec,
            pl.BlockSpec((C, C), lambda i: (0, 0)),
            vec, vec,
        ],
        out_specs=pl.BlockSpec((BM, C), lambda i: (i, 0)),
        out_shape=jax.ShapeDtypeStruct((M, C), jnp.float32),
    )(S, dpre, W3, g2.reshape(1, C), b2.reshape(1, C), g3.reshape(1, C),
      b3.reshape(1, C), Wd, gd.reshape(1, C), bd.reshape(1, C))


def kernel(x, W1, g1, b1, W2, g2, b2, W3, g3, b3, Wd, gd, bd,
           in_idx, out_idx, koff, ds_idx):
    in_idx = in_idx.astype(jnp.int32)
    out_idx = out_idx.astype(jnp.int32)
    koff = koff.astype(jnp.int32)
    ds_idx = ds_idx.astype(jnp.int32)

    xw = _stage_a(x, W1, g1, b1, W2)
    xw2 = xw.reshape(KN, C)
    P, counts = _stage_p(in_idx, koff, out_idx)
    Ppart = _reorder_kernel(P)
    zeros_init = jnp.zeros((ASUB, C), jnp.float32)
    S = _edge_kernel(Ppart, counts, xw2, zeros_init)
    dpre = _ds_kernel(ds_idx, x)
    return _stage_c(S, dpre, W3, g2, b2, g3, b3, Wd, gd, bd)


# speedup vs baseline: 65.5674x; 1.1140x over previous
import dataclasses
import functools

import jax
import jax.numpy as jnp
from jax import lax
from jax.experimental import pallas as pl
from jax.experimental.pallas import tpu as pltpu
from jax.experimental.pallas import tpu_sc as plsc

N = 50000
M = 25000
C = 128
E = 400000
K = 27
G = 8
KN = K * N

BN = 400
BM = 1000
EW = 128
NWINP = 3136
EP = NWINP * EW
BW = 112
EB = BW * EW
EW2 = 64
NWIN2 = EP // EW2
TSUB = NWIN2 // 16
MH = 12800
AR = MH + 64
ASUB = AR // 16
OSUB = MH // 16
DW = 40
NDWIN = M // DW
NB = 2
NBE = 3
_EPS = 1e-5

_SC_MESH = dict(core_axis_name="c", subcore_axis_name="s")


def _sc_params():
    cp = pltpu.CompilerParams()
    return dataclasses.replace(cp, needs_layout_passes=False,
                               use_tc_tiling_on_sc=False)


def _gn_tc(h, gamma, beta, groups):
    n, c = h.shape
    cs = c // groups
    gi = lax.broadcasted_iota(jnp.int32, (c, groups), 0) // cs
    gj = lax.broadcasted_iota(jnp.int32, (c, groups), 1)
    Gm = (gi == gj).astype(jnp.float32)
    ti = lax.broadcasted_iota(jnp.int32, (groups, c), 0)
    tj = lax.broadcasted_iota(jnp.int32, (groups, c), 1) // cs
    GmT = (ti == tj).astype(jnp.float32)
    inv_cs = 1.0 / cs
    mu = jnp.dot(h, Gm, preferred_element_type=jnp.float32) * inv_cs
    ex2 = jnp.dot(h * h, Gm, preferred_element_type=jnp.float32) * inv_cs
    var = ex2 - mu * mu
    rstd = lax.rsqrt(var + _EPS)
    mub = jnp.dot(mu, GmT, preferred_element_type=jnp.float32)
    rstdb = jnp.dot(rstd, GmT, preferred_element_type=jnp.float32)
    return (h - mub) * rstdb * gamma + beta


def _stage_a_body(x_ref, w1_ref, g1_ref, b1_ref, w2_ref, xw_ref):
    h = jnp.dot(x_ref[...], w1_ref[...], preferred_element_type=jnp.float32)
    h = _gn_tc(h, g1_ref[...], b1_ref[...], G)
    h = jnp.where(h >= 0, h, 0.01 * h)
    for k in range(K):
        xw_ref[k] = jnp.dot(h, w2_ref[k], preferred_element_type=jnp.float32)


def _stage_a(x, W1, g1, b1, W2):
    return pl.pallas_call(
        _stage_a_body,
        grid=(N // BN,),
        in_specs=[
            pl.BlockSpec((BN, C), lambda i: (i, 0)),
            pl.BlockSpec((C, C), lambda i: (0, 0)),
            pl.BlockSpec((1, C), lambda i: (0, 0)),
            pl.BlockSpec((1, C), lambda i: (0, 0)),
            pl.BlockSpec((K, C, C), lambda i: (0, 0, 0)),
        ],
        out_specs=pl.BlockSpec((K, BN, C), lambda i: (0, i, 0)),
        out_shape=jax.ShapeDtypeStruct((K, N, C), jnp.float32),
    )(x, W1, g1.reshape(1, C), b1.reshape(1, C), W2)


def _tri_lanes(n):
    a = lax.broadcasted_iota(jnp.int32, (n, n), 0)
    b = lax.broadcasted_iota(jnp.int32, (n, n), 1)
    return (a < b).astype(jnp.float32)


def _tri_rows(n):
    a = lax.broadcasted_iota(jnp.int32, (n, n), 0)
    b = lax.broadcasted_iota(jnp.int32, (n, n), 1)
    return (b < a).astype(jnp.float32)


def _stage_p_body(ii_ref, kf_ref, oi_ref, p_ref, cnt_ref, carry):
    i = pl.program_id(0)

    @pl.when(i == 0)
    def _():
        carry[0] = 0
        carry[1] = 0

    ii = ii_ref[...]
    kf = kf_ref[...]
    oi = oi_ref[...]
    row = lax.broadcasted_iota(jnp.int32, (BW, EW), 0)
    lane = lax.broadcasted_iota(jnp.int32, (BW, EW), 1)
    eidx = i * EB + row * EW + lane
    is_pad = eidx >= E
    gidx = jnp.where(is_pad, lane, kf * jnp.int32(N) + ii)

    c0 = carry[0]
    c1 = carry[1]
    b1r = (oi >= MH).astype(jnp.int32)
    s0_real = jnp.sum(jnp.where(is_pad, 0, 1 - b1r))
    n0_real_total = c0 + s0_real
    padn0 = (-n0_real_total) % jnp.int32(EW)
    pad_rank = eidx - jnp.int32(E)
    b1pad = (pad_rank >= padn0).astype(jnp.int32)
    b1i = jnp.where(is_pad, b1pad, b1r)

    trash = jnp.int32(MH) + (lane & 63)
    li = jnp.where(is_pad, trash,
                   jnp.where(b1i == 1, oi - jnp.int32(MH), oi))

    b1f = b1i.astype(jnp.float32)
    excl1 = jnp.dot(b1f, _tri_lanes(EW), preferred_element_type=jnp.float32)
    rs1 = jnp.dot(b1f, jnp.ones((EW, 1), jnp.float32),
                  preferred_element_type=jnp.float32)
    rp1 = jnp.dot(_tri_rows(BW), rs1, preferred_element_type=jnp.float32)
    pos1 = (rp1 + excl1).astype(jnp.int32)
    iflat = row * EW + lane
    pos0 = iflat - pos1
    dst = jnp.where(b1i == 1, jnp.int32(EP - 1) - (c1 + pos1), c0 + pos0)

    p_ref[0] = gidx
    p_ref[1] = li
    p_ref[2] = dst

    s1 = jnp.sum(b1i)
    s0 = jnp.int32(EB) - s1
    carry[0] = c0 + s0
    carry[1] = c1 + s1

    @pl.when(i == (NWINP // BW) - 1)
    def _():
        nwin0 = (c0 + s0) // jnp.int32(EW2)
        l2 = lax.broadcasted_iota(jnp.int32, (1, C), 1)
        cnt_ref[...] = jnp.where(
            l2 == 0, nwin0,
            jnp.where(l2 == 1, jnp.int32(NWIN2) - nwin0, 0))


def _stage_p(in_idx, koff, out_idx):
    ii = jnp.pad(in_idx, (0, EP - E)).reshape(NWINP, EW)
    kf = jnp.pad(koff, (0, EP - E)).reshape(NWINP, EW)
    oi = jnp.pad(out_idx, (0, EP - E)).reshape(NWINP, EW)
    blk = pl.BlockSpec((BW, EW), lambda i: (i, 0))
    return pl.pallas_call(
        _stage_p_body,
        grid=(NWINP // BW,),
        in_specs=[blk, blk, blk],
        out_specs=[
            pl.BlockSpec((3, BW, EW), lambda i: (0, i, 0)),
            pl.BlockSpec((1, C), lambda i: (0, 0)),
        ],
        out_shape=[
            jax.ShapeDtypeStruct((3, NWINP, EW), jnp.int32),
            jax.ShapeDtypeStruct((1, C), jnp.int32),
        ],
        scratch_shapes=[pltpu.SMEM((2,), jnp.int32)],
    )(ii, kf, oi)


def _reorder_kernel(P):
    mesh = plsc.VectorSubcoreMesh(**_SC_MESH)

    @functools.partial(
        pl.kernel,
        out_type=jax.ShapeDtypeStruct((EP, 16), jnp.int32),
        mesh=mesh,
        compiler_params=_sc_params(),
        scratch_types=[
            pltpu.VMEM((NB, EW), jnp.int32),
            pltpu.VMEM((NB, EW), jnp.int32),
            pltpu.VMEM((NB, EW), jnp.int32),
            pltpu.VMEM((NB, EW, 16), jnp.int32),
        ] + [pltpu.SemaphoreType.DMA] * (2 * NB),
    )
    def body(p_hbm, o_hbm, gi_v, li_v, di_v, pe, *sems):
        isem = sems[:NB]
        ssem = sems[NB:]
        c = lax.axis_index("c")
        s = lax.axis_index("s")
        wid = s * 2 + c
        rows16 = lax.iota(jnp.int32, 16)
        z16 = jnp.zeros((16,), jnp.int32)

        @pl.loop(0, NWINP // 32 // NB)
        def _it(u):
            for j in range(NB):
                @pl.when(u > 0)
                def _():
                    pltpu.make_async_copy(pe.at[j], o_hbm.at[di_v.at[j]],
                                          ssem[j]).wait()
                w = wid + (u * NB + j) * 32
                pltpu.async_copy(p_hbm.at[0].at[w], gi_v.at[j], isem[j])
                pltpu.async_copy(p_hbm.at[1].at[w], li_v.at[j], isem[j])
                pltpu.async_copy(p_hbm.at[2].at[w], di_v.at[j], isem[j])
            for j in range(NB):
                w = wid + (u * NB + j) * 32
                pltpu.make_async_copy(p_hbm.at[0].at[w],
                                      gi_v.at[j], isem[j]).wait()
                pltpu.make_async_copy(p_hbm.at[1].at[w],
                                      li_v.at[j], isem[j]).wait()
                pltpu.make_async_copy(p_hbm.at[2].at[w],
                                      di_v.at[j], isem[j]).wait()
                for t in range(EW // 16):
                    sl = pl.ds(t * 16, 16)
                    plsc.store_scatter(pe.at[j], [rows16 + t * 16, z16],
                                       gi_v[j, sl])
                    plsc.store_scatter(pe.at[j], [rows16 + t * 16, z16 + 1],
                                       li_v[j, sl])
                pltpu.async_copy(pe.at[j], o_hbm.at[di_v.at[j]], ssem[j])

        for j in range(NB):
            pltpu.make_async_copy(pe.at[j], o_hbm.at[di_v.at[j]],
                                  ssem[j]).wait()

    return body(P)


def _edge_kernel(Ppart, counts, xw2, zeros_init):
    mesh = plsc.VectorSubcoreMesh(**_SC_MESH)
    TSUB = NWIN2 // 16

    @functools.partial(
        pl.kernel,
        out_type=jax.ShapeDtypeStruct((2, MH, C), jnp.float32),
        mesh=mesh,
        compiler_params=_sc_params(),
        scratch_types=[
            pltpu.VMEM((16,), jnp.int32),
            pltpu.VMEM((NBE, EW2, 16), jnp.int32),
            pltpu.VMEM((NBE, EW2), jnp.int32),
            pltpu.VMEM((NBE, EW2), jnp.int32),
            pltpu.VMEM((NBE, EW2, C), jnp.float32),
            pltpu.VMEM_SHARED((AR, C), jnp.float32),
        ] + [pltpu.SemaphoreType.DMA] * (3 * NBE),
    )
    def body(pp_hbm, cnt_hbm, xw_hbm, z_hbm, out_hbm,
             cnt_v, pb, gi_v, li_v, rows, acc, *sems):
        isem = sems[:NBE]
        gsem = sems[NBE:2 * NBE]
        ssem = sems[2 * NBE:]
        c = lax.axis_index("c")
        s = lax.axis_index("s")
        pltpu.sync_copy(z_hbm, acc.at[pl.ds(s * ASUB, ASUB)])
        pltpu.sync_copy(cnt_hbm.at[0].at[pl.ds(0, 16)], cnt_v)
        lane16 = lax.iota(jnp.int32, 16)
        nwin = jnp.sum(jnp.where(lane16 == c, cnt_v[...], 0))
        plsc.subcore_barrier()
        rows16 = lax.iota(jnp.int32, 16)
        z16 = jnp.zeros((16,), jnp.int32)

        @pl.loop(0, 131)
        def _it(u):
            for j in range(NBE):
                w = s + (u * NBE + j) * 16

                @pl.when(w < nwin)
                def _():
                    @pl.when(u > 0)
                    def _():
                        pltpu.make_async_copy(
                            rows.at[j], acc.at[li_v.at[j]], ssem[j]).wait()
                    base = jnp.where(c == 0, w * EW2,
                                     jnp.int32(EP) - (w + 1) * EW2)
                    pltpu.async_copy(pp_hbm.at[pl.ds(base, EW2)], pb.at[j],
                                     isem[j])
            for j in range(NBE):
                w = s + (u * NBE + j) * 16

                @pl.when(w < nwin)
                def _():
                    pltpu.make_async_copy(pp_hbm.at[pl.ds(0, EW2)], pb.at[j],
                                          isem[j]).wait()
                    for t in range(EW2 // 16):
                        sl = pl.ds(t * 16, 16)
                        gi_v[j, sl] = plsc.load_gather(
                            pb.at[j], [rows16 + t * 16, z16])
                        li_v[j, sl] = plsc.load_gather(
                            pb.at[j], [rows16 + t * 16, z16 + 1])
                    pltpu.async_copy(xw_hbm.at[gi_v.at[j]], rows.at[j],
                                     gsem[j])
            for j in range(NBE):
                w = s + (u * NBE + j) * 16

                @pl.when(w < nwin)
                def _():
                    pltpu.make_async_copy(xw_hbm.at[gi_v.at[j]], rows.at[j],
                                          gsem[j]).wait()
                    pltpu.async_copy(rows.at[j], acc.at[li_v.at[j]], ssem[j],
                                     add=True)

        for j in range(NBE):
            @pl.when((s + j * 16) < nwin)
            def _():
                pltpu.make_async_copy(rows.at[j], acc.at[li_v.at[j]],
                                      ssem[j]).wait()

        plsc.subcore_barrier()
        pltpu.sync_copy(acc.at[pl.ds(s * OSUB, OSUB)],
                        out_hbm.at[c].at[pl.ds(s * OSUB, OSUB)])

    return body(Ppart, counts, xw2, zeros_init)


def _ds_kernel(ds_idx, x):
    mesh = plsc.VectorSubcoreMesh(**_SC_MESH)

    @functools.partial(
        pl.kernel,
        out_type=jax.ShapeDtypeStruct((M, C), jnp.float32),
        mesh=mesh,
        scratch_types=[
            pltpu.VMEM((DW,), jnp.int32),
            pltpu.VMEM((DW, C), jnp.float32),
        ],
    )
    def body(di_hbm, x_hbm, out_hbm, di_v, rows_v):
        c = lax.axis_index("c")
        s = lax.axis_index("s")
        wid = s * 2 + c

        @pl.loop(0, 20)
        def _win(t):
            w = wid + t * 32

            @pl.when(w < NDWIN)
            def _():
                base = w * DW
                pltpu.sync_copy(di_hbm.at[pl.ds(base, DW)], di_v)
                pltpu.sync_copy(x_hbm.at[di_v], rows_v)
                pltpu.sync_copy(rows_v, out_hbm.at[pl.ds(base, DW)])

    return body(ds_idx, x)


def _stage_c_body(s_ref, dpre_ref, w3_ref, g2_ref, b2_ref, g3_ref, b3_ref,
                  wd_ref, gd_ref, bd_ref, out_ref):
    t = _gn_tc(s_ref[...], g2_ref[...], b2_ref[...], G)
    u = jnp.dot(t, w3_ref[...], preferred_element_type=jnp.float32)
    u = _gn_tc(u, g3_ref[...], b3_ref[...], G)
    d = jnp.dot(dpre_ref[...], wd_ref[...], preferred_element_type=jnp.float32)
    d = _gn_tc(d, gd_ref[...], bd_ref[...], G)
    out_ref[...] = u + d


def _stage_c(S, dpre, W3, g2, b2, g3, b3, Wd, gd, bd):
    S = S.reshape(2 * MH, C)
    vec = pl.BlockSpec((1, C), lambda i: (0, 0))
    return pl.pallas_call(
        _stage_c_body,
        grid=(M // BM,),
        in_specs=[
            pl.BlockSpec((BM, C), lambda i: (i, 0)),
            pl.BlockSpec((BM, C), lambda i: (i, 0)),
            pl.BlockSpec((C, C), lambda i: (0, 0)),
            vec, vec, vec, vec,
            pl.BlockSpec((C, C), lambda i: (0, 0)),
            vec, vec,
        ],
        out_specs=pl.BlockSpec((BM, C), lambda i: (i, 0)),
        out_shape=jax.ShapeDtypeStruct((M, C), jnp.float32),
    )(S, dpre, W3, g2.reshape(1, C), b2.reshape(1, C), g3.reshape(1, C),
      b3.reshape(1, C), Wd, gd.reshape(1, C), bd.reshape(1, C))


def kernel(x, W1, g1, b1, W2, g2, b2, W3, g3, b3, Wd, gd, bd,
           in_idx, out_idx, koff, ds_idx):
    in_idx = in_idx.astype(jnp.int32)
    out_idx = out_idx.astype(jnp.int32)
    koff = koff.astype(jnp.int32)
    ds_idx = ds_idx.astype(jnp.int32)

    P, counts = _stage_p(in_idx, koff, out_idx)
    Ppart = _reorder_kernel(P)
    dpre = _ds_kernel(ds_idx, x)
    xw = _stage_a(x, W1, g1, b1, W2)
    xw2 = xw.reshape(KN, C)
    zeros_init = jnp.zeros((ASUB, C), jnp.float32)
    S = _edge_kernel(Ppart, counts, xw2, zeros_init)
    return _stage_c(S, dpre, W3, g2, b2, g3, b3, Wd, gd, bd)
